# baseline (speedup 1.0000x reference)
import jax
import jax.numpy as jnp
from jax import lax
from jax.experimental import pallas as pl
from jax.experimental.pallas import tpu as pltpu
from jax.experimental.pallas import tpu_sc as plsc

NW = 32
NC = 2
L = 16
B = 128
G = B // NW
QSTR = 17

N1, PPG1, EPG1 = 32, 256, 64
N2, PPG2, EPG2 = 64, 512, 128
E1 = B * EPG1
E2 = B * EPG2


def _qwords(epg):
    return ((G * epg + 1) * QSTR + L - 1) // L * L


def _memset(ref, nvec, vec):
    def body(i, c):
        base = i * (4 * L)
        for j in range(4):
            ref[pl.ds(base + j * L, L)] = vec
        return c
    lax.fori_loop(0, nvec // 4, body, 0)
    for j in range(nvec // 4 * 4, nvec):
        ref[pl.ds(j * L, L)] = vec


def _half_compute(N, ppg, epg, out_h,
                  psrc, pdst, plen, esrc, edst, smap, q, qv, wid, sem):
    npr = G * ppg
    ned = G * epg
    nself = G * N
    cells = G * N * N
    ebase = wid * ned
    cell_off = wid * cells
    iota = lax.iota(jnp.int32, L)
    ones = jnp.ones((L,), jnp.float32)
    mask_n = N - 1

    def ekey(i):
        s = esrc[pl.ds(i * L, L)]
        d = edst[pl.ds(i * L, L)]
        return s * N + (d & mask_n) - cell_off

    def scat_e(i, c):
        plsc.store_scatter(smap, [ekey(i)], i * L + iota)
        return c

    lax.fori_loop(0, ned // L, scat_e, 0)

    def pair_step(i, c):
        s = psrc[pl.ds(i * L, L)]
        d = pdst[pl.ds(i * L, L)]
        t = plen[pl.ds(i * L, L)] + 1
        k = s * N + (d & mask_n) - cell_off
        slot = plsc.load_gather(smap, [k])
        plsc.addupdate_scatter(q, [slot * QSTR + t], ones)
        return c

    lax.fori_loop(0, npr // L, pair_step, 0)

    def self_step(i, c):
        iloc = i * L + iota
        k = iloc * N + (iloc & mask_n)
        slot = plsc.load_gather(smap, [k])
        plsc.addupdate_scatter(q, [slot * QSTR], ones)
        return c

    lax.fori_loop(0, nself // L, self_step, 0)

    def out_step(i, c):
        slot = plsc.load_gather(smap, [ekey(i)]) * QSTR
        for t in range(L):
            vals = plsc.load_gather(q, [slot + t])
            qv[t, pl.ds(i * L, L)] = vals
        return c

    lax.fori_loop(0, ned // L, out_step, 0)

    return pltpu.async_copy(qv, out_h.at[:, pl.ds(ebase, ned)], sem)


def _sc_body(pidx1_h, plen1_h, eidx1_h,
             pidx2_h, plen2_h, eidx2_h,
             q1_out, q2_out,
             psrc1, pdst1, plen1, esrc1, edst1, smap1, q1, qv1,
             psrc2, pdst2, plen2, esrc2, edst2, smap2, q2, qv2,
             sem):
    wid = lax.axis_index("s") * NC + lax.axis_index("c")
    npr1, ned1 = G * PPG1, G * EPG1
    npr2, ned2 = G * PPG2, G * EPG2

    cps = []
    for hbm, row, vmem, base, n in (
            (pidx1_h, 0, psrc1, wid * npr1, npr1),
            (pidx1_h, 1, pdst1, wid * npr1, npr1),
            (plen1_h, None, plen1, wid * npr1, npr1),
            (eidx1_h, 0, esrc1, wid * ned1, ned1),
            (eidx1_h, 1, edst1, wid * ned1, ned1),
            (pidx2_h, 0, psrc2, wid * npr2, npr2),
            (pidx2_h, 1, pdst2, wid * npr2, npr2),
            (plen2_h, None, plen2, wid * npr2, npr2),
            (eidx2_h, 0, esrc2, wid * ned2, ned2),
            (eidx2_h, 1, edst2, wid * ned2, ned2)):
        src = hbm.at[pl.ds(base, n)] if row is None \
            else hbm.at[row, pl.ds(base, n)]
        cps.append(pltpu.async_copy(src, vmem, sem))

    _memset(smap1, G * N1 * N1 // L, jnp.full((L,), ned1, jnp.int32))
    _memset(smap2, G * N2 * N2 // L, jnp.full((L,), ned2, jnp.int32))
    zf = jnp.zeros((L,), jnp.float32)
    _memset(q1, _qwords(EPG1) // L, zf)
    _memset(q2, _qwords(EPG2) // L, zf)

    for cp in cps:
        cp.wait()

    ocp1 = _half_compute(N1, PPG1, EPG1, q1_out,
                         psrc1, pdst1, plen1, esrc1, edst1, smap1, q1, qv1,
                         wid, sem)
    ocp2 = _half_compute(N2, PPG2, EPG2, q2_out,
                         psrc2, pdst2, plen2, esrc2, edst2, smap2, q2, qv2,
                         wid, sem)
    ocp1.wait()
    ocp2.wait()


def _half_scratch(N, ppg, epg):
    npr = G * ppg
    ned = G * epg
    cells = G * N * N
    return [
        pltpu.VMEM((npr,), jnp.int32),
        pltpu.VMEM((npr,), jnp.int32),
        pltpu.VMEM((npr,), jnp.int32),
        pltpu.VMEM((ned,), jnp.int32),
        pltpu.VMEM((ned,), jnp.int32),
        pltpu.VMEM((cells,), jnp.int32),
        pltpu.VMEM((_qwords(epg),), jnp.float32),
        pltpu.VMEM((L, ned), jnp.float32),
    ]


_sc_counts = pl.kernel(
    _sc_body,
    out_type=(jax.ShapeDtypeStruct((L, E1), jnp.float32),
              jax.ShapeDtypeStruct((L, E2), jnp.float32)),
    mesh=plsc.VectorSubcoreMesh(core_axis_name="c", subcore_axis_name="s"),
    scratch_types=_half_scratch(N1, PPG1, EPG1) + _half_scratch(N2, PPG2, EPG2)
    + [pltpu.SemaphoreType.DMA],
    compiler_params=pltpu.CompilerParams(needs_layout_passes=False),
)


def _tc_body(q1_ref, w1_ref, q2_ref, w2_ref, o1_ref, o2_ref):
    dn = (((0,), (0,)), ((), ()))
    w1 = jnp.pad(w1_ref[...], ((0, L - 12), (0, 0)))
    w2 = jnp.pad(w2_ref[...], ((0, L - 12), (0, 0)))
    o1_ref[...] = lax.dot_general(q1_ref[...], w1, dn,
                                  preferred_element_type=jnp.float32)
    o2_ref[...] = lax.dot_general(q2_ref[...], w2, dn,
                                  preferred_element_type=jnp.float32)


_TCG = 8
_B1 = E1 // _TCG
_B2 = E2 // _TCG

_tc_project = pl.pallas_call(
    _tc_body,
    grid=(_TCG,),
    in_specs=[
        pl.BlockSpec((L, _B1), lambda i: (0, i)),
        pl.BlockSpec((12, 64), lambda i: (0, 0)),
        pl.BlockSpec((L, _B2), lambda i: (0, i)),
        pl.BlockSpec((12, 64), lambda i: (0, 0)),
    ],
    out_specs=(pl.BlockSpec((_B1, 64), lambda i: (i, 0)),
               pl.BlockSpec((_B2, 64), lambda i: (i, 0))),
    out_shape=(jax.ShapeDtypeStruct((E1, 64), jnp.float32),
               jax.ShapeDtypeStruct((E2, 64), jnp.float32)),
)


def kernel(spd_index, spd_lengths, batch, edge_index,
           e2e_spd_index, e2e_spd_lengths, e_batch, e2e_edge_index,
           W_spd, W_e2e):
    del batch, e_batch
    q1, q2 = _sc_counts(
        spd_index, spd_lengths, edge_index,
        e2e_spd_index, e2e_spd_lengths, e2e_edge_index,
    )
    return _tc_project(q1, W_spd, q2, W_e2e)


# speedup vs baseline: 1.0452x; 1.0452x over previous
import jax
import jax.numpy as jnp
from jax import lax
from jax.experimental import pallas as pl
from jax.experimental.pallas import tpu as pltpu
from jax.experimental.pallas import tpu_sc as plsc

NW = 32
NC = 2
L = 16
B = 128
G = B // NW
QSTR = 17

N1, PPG1, EPG1 = 32, 256, 64
N2, PPG2, EPG2 = 64, 512, 128
E1 = B * EPG1
E2 = B * EPG2


def _qwords(epg):
    return ((G * epg + 1) * QSTR + L - 1) // L * L


def _memset(ref, nvec, vec):
    def body(i, c):
        base = i * (4 * L)
        for j in range(4):
            ref[pl.ds(base + j * L, L)] = vec
        return c
    lax.fori_loop(0, nvec // 4, body, 0)
    for j in range(nvec // 4 * 4, nvec):
        ref[pl.ds(j * L, L)] = vec


def _half_compute(N, ppg, epg, out_h,
                  psrc, pdst, plen, esrc, edst, smap, q, qv, wid, sem):
    npr = G * ppg
    ned = G * epg
    nself = G * N
    cells = G * N * N
    ebase = wid * ned
    cell_off = wid * cells
    iota = lax.iota(jnp.int32, L)
    ones = jnp.ones((L,), jnp.float32)
    mask_n = N - 1

    def ekey(i):
        s = esrc[pl.ds(i * L, L)]
        d = edst[pl.ds(i * L, L)]
        return s * N + (d & mask_n) - cell_off

    def scat_e(i, c):
        plsc.store_scatter(smap, [ekey(i)], i * L + iota)
        return c

    lax.fori_loop(0, ned // L, scat_e, 0)

    def pair_step(i, c):
        s = psrc[pl.ds(i * L, L)]
        d = pdst[pl.ds(i * L, L)]
        t = plen[pl.ds(i * L, L)] + 1
        k = s * N + (d & mask_n) - cell_off
        slot = plsc.load_gather(smap, [k])
        plsc.addupdate_scatter(q, [slot * QSTR + t], ones)
        return c

    lax.fori_loop(0, npr // L, pair_step, 0)

    def self_step(i, c):
        iloc = i * L + iota
        k = iloc * N + (iloc & mask_n)
        slot = plsc.load_gather(smap, [k])
        plsc.addupdate_scatter(q, [slot * QSTR], ones)
        return c

    lax.fori_loop(0, nself // L, self_step, 0)

    def out_step(i, c):
        slot = plsc.load_gather(smap, [ekey(i)]) * QSTR
        for t in range(L):
            vals = plsc.load_gather(q, [slot + t])
            qv[t, pl.ds(i * L, L)] = vals
        return c

    lax.fori_loop(0, ned // L, out_step, 0)

    return pltpu.async_copy(qv, out_h.at[:, pl.ds(ebase, ned)], sem)


def _sc_body(pidx1_h, plen1_h, eidx1_h,
             pidx2_h, plen2_h, eidx2_h,
             q1_out, q2_out,
             psrc1, pdst1, plen1, esrc1, edst1, smap1, q1, qv1,
             psrc2, pdst2, plen2, esrc2, edst2, smap2, q2, qv2,
             sem):
    wid = lax.axis_index("s") * NC + lax.axis_index("c")
    npr1, ned1 = G * PPG1, G * EPG1
    npr2, ned2 = G * PPG2, G * EPG2

    cps = []
    for hbm, row, vmem, base, n in (
            (pidx1_h, 0, psrc1, wid * npr1, npr1),
            (pidx1_h, 1, pdst1, wid * npr1, npr1),
            (plen1_h, None, plen1, wid * npr1, npr1),
            (eidx1_h, 0, esrc1, wid * ned1, ned1),
            (eidx1_h, 1, edst1, wid * ned1, ned1),
            (pidx2_h, 0, psrc2, wid * npr2, npr2),
            (pidx2_h, 1, pdst2, wid * npr2, npr2),
            (plen2_h, None, plen2, wid * npr2, npr2),
            (eidx2_h, 0, esrc2, wid * ned2, ned2),
            (eidx2_h, 1, edst2, wid * ned2, ned2)):
        src = hbm.at[pl.ds(base, n)] if row is None \
            else hbm.at[row, pl.ds(base, n)]
        cps.append(pltpu.async_copy(src, vmem, sem))

    _memset(smap1, G * N1 * N1 // L, jnp.full((L,), ned1, jnp.int32))
    _memset(smap2, G * N2 * N2 // L, jnp.full((L,), ned2, jnp.int32))
    zf = jnp.zeros((L,), jnp.float32)
    _memset(q1, _qwords(EPG1) // L, zf)
    _memset(q2, _qwords(EPG2) // L, zf)

    for cp in cps:
        cp.wait()

    ocp1 = _half_compute(N1, PPG1, EPG1, q1_out,
                         psrc1, pdst1, plen1, esrc1, edst1, smap1, q1, qv1,
                         wid, sem)
    ocp2 = _half_compute(N2, PPG2, EPG2, q2_out,
                         psrc2, pdst2, plen2, esrc2, edst2, smap2, q2, qv2,
                         wid, sem)
    ocp1.wait()
    ocp2.wait()


def _half_scratch(N, ppg, epg):
    npr = G * ppg
    ned = G * epg
    cells = G * N * N
    return [
        pltpu.VMEM((npr,), jnp.int32),
        pltpu.VMEM((npr,), jnp.int32),
        pltpu.VMEM((npr,), jnp.int32),
        pltpu.VMEM((ned,), jnp.int32),
        pltpu.VMEM((ned,), jnp.int32),
        pltpu.VMEM((cells,), jnp.int32),
        pltpu.VMEM((_qwords(epg),), jnp.float32),
        pltpu.VMEM((L, ned), jnp.float32),
    ]


_sc_counts = pl.kernel(
    _sc_body,
    out_type=(jax.ShapeDtypeStruct((L, E1), jnp.float32),
              jax.ShapeDtypeStruct((L, E2), jnp.float32)),
    mesh=plsc.VectorSubcoreMesh(core_axis_name="c", subcore_axis_name="s"),
    scratch_types=_half_scratch(N1, PPG1, EPG1) + _half_scratch(N2, PPG2, EPG2)
    + [pltpu.SemaphoreType.DMA],
    compiler_params=pltpu.CompilerParams(needs_layout_passes=False),
)


def _tc_body(q1_ref, w1_ref, q2_ref, w2_ref, o1_ref, o2_ref):
    dn = (((0,), (0,)), ((), ()))
    w1 = jnp.pad(w1_ref[...], ((0, L - 12), (0, 0)))
    w2 = jnp.pad(w2_ref[...], ((0, L - 12), (0, 0)))
    o1_ref[...] = lax.dot_general(q1_ref[...], w1, dn,
                                  preferred_element_type=jnp.float32)
    o2_ref[...] = lax.dot_general(q2_ref[...], w2, dn,
                                  preferred_element_type=jnp.float32)


_TCG = 4
_B1 = E1 // _TCG
_B2 = E2 // _TCG

_tc_project = pl.pallas_call(
    _tc_body,
    grid=(_TCG,),
    in_specs=[
        pl.BlockSpec((L, _B1), lambda i: (0, i)),
        pl.BlockSpec((12, 64), lambda i: (0, 0)),
        pl.BlockSpec((L, _B2), lambda i: (0, i)),
        pl.BlockSpec((12, 64), lambda i: (0, 0)),
    ],
    out_specs=(pl.BlockSpec((_B1, 64), lambda i: (i, 0)),
               pl.BlockSpec((_B2, 64), lambda i: (i, 0))),
    out_shape=(jax.ShapeDtypeStruct((E1, 64), jnp.float32),
               jax.ShapeDtypeStruct((E2, 64), jnp.float32)),
)


def kernel(spd_index, spd_lengths, batch, edge_index,
           e2e_spd_index, e2e_spd_lengths, e_batch, e2e_edge_index,
           W_spd, W_e2e):
    del batch, e_batch
    q1, q2 = _sc_counts(
        spd_index, spd_lengths, edge_index,
        e2e_spd_index, e2e_spd_lengths, e2e_edge_index,
    )
    return _tc_project(q1, W_spd, q2, W_e2e)


# speedup vs baseline: 1.0566x; 1.0109x over previous
import jax
import jax.numpy as jnp
from jax import lax
from jax.experimental import pallas as pl
from jax.experimental.pallas import tpu as pltpu
from jax.experimental.pallas import tpu_sc as plsc

NW = 32
NC = 2
L = 16
B = 128
G = B // NW
QSTR = 17

N1, PPG1, EPG1 = 32, 256, 64
N2, PPG2, EPG2 = 64, 512, 128
E1 = B * EPG1
E2 = B * EPG2


def _qwords(epg):
    return ((G * epg + 1) * QSTR + L - 1) // L * L


def _memset(ref, nvec, vec):
    def body(i, c):
        base = i * (4 * L)
        for j in range(4):
            ref[pl.ds(base + j * L, L)] = vec
        return c
    lax.fori_loop(0, nvec // 4, body, 0)
    for j in range(nvec // 4 * 4, nvec):
        ref[pl.ds(j * L, L)] = vec


def _half_compute(N, ppg, epg, out_h,
                  psrc, pdst, plen, esrc, edst, smap, q, qv, wid, sem):
    npr = G * ppg
    ned = G * epg
    nself = G * N
    cells = G * N * N
    ebase = wid * ned
    cell_off = wid * cells
    iota = lax.iota(jnp.int32, L)
    ones = jnp.ones((L,), jnp.float32)
    mask_n = N - 1

    def ekey(i):
        s = esrc[pl.ds(i * L, L)]
        d = edst[pl.ds(i * L, L)]
        return s * N + (d & mask_n) - cell_off

    def scat_e(i, c):
        plsc.store_scatter(smap, [ekey(i)], i * L + iota)
        return c

    lax.fori_loop(0, ned // L, scat_e, 0)

    def pair_step(i, c):
        s = psrc[pl.ds(i * L, L)]
        d = pdst[pl.ds(i * L, L)]
        t = plen[pl.ds(i * L, L)] + 1
        k = s * N + (d & mask_n) - cell_off
        slot = plsc.load_gather(smap, [k])
        plsc.addupdate_scatter(q, [slot * QSTR + t], ones)
        return c

    lax.fori_loop(0, npr // L, pair_step, 0)

    def self_step(i, c):
        iloc = i * L + iota
        k = iloc * N + (iloc & mask_n)
        slot = plsc.load_gather(smap, [k])
        plsc.addupdate_scatter(q, [slot * QSTR], ones)
        return c

    lax.fori_loop(0, nself // L, self_step, 0)

    def out_step(i, c):
        slot = plsc.load_gather(smap, [ekey(i)]) * QSTR
        for t in range(L):
            vals = plsc.load_gather(q, [slot + t])
            qv[t, pl.ds(i * L, L)] = vals
        return c

    lax.fori_loop(0, ned // L, out_step, 0)

    return pltpu.async_copy(qv, out_h.at[:, pl.ds(ebase, ned)], sem)


def _sc_body(pidx1_h, plen1_h, eidx1_h,
             pidx2_h, plen2_h, eidx2_h,
             q1_out, q2_out,
             psrc1, pdst1, plen1, esrc1, edst1, smap1, q1, qv1,
             psrc2, pdst2, plen2, esrc2, edst2, smap2, q2, qv2,
             sem):
    wid = lax.axis_index("s") * NC + lax.axis_index("c")
    npr1, ned1 = G * PPG1, G * EPG1
    npr2, ned2 = G * PPG2, G * EPG2

    cps = []
    for hbm, row, vmem, base, n in (
            (pidx1_h, 0, psrc1, wid * npr1, npr1),
            (pidx1_h, 1, pdst1, wid * npr1, npr1),
            (plen1_h, None, plen1, wid * npr1, npr1),
            (eidx1_h, 0, esrc1, wid * ned1, ned1),
            (eidx1_h, 1, edst1, wid * ned1, ned1),
            (pidx2_h, 0, psrc2, wid * npr2, npr2),
            (pidx2_h, 1, pdst2, wid * npr2, npr2),
            (plen2_h, None, plen2, wid * npr2, npr2),
            (eidx2_h, 0, esrc2, wid * ned2, ned2),
            (eidx2_h, 1, edst2, wid * ned2, ned2)):
        src = hbm.at[pl.ds(base, n)] if row is None \
            else hbm.at[row, pl.ds(base, n)]
        cps.append(pltpu.async_copy(src, vmem, sem))

    _memset(smap1, G * N1 * N1 // L, jnp.full((L,), ned1, jnp.int32))
    _memset(smap2, G * N2 * N2 // L, jnp.full((L,), ned2, jnp.int32))
    zf = jnp.zeros((L,), jnp.float32)
    _memset(q1, _qwords(EPG1) // L, zf)
    _memset(q2, _qwords(EPG2) // L, zf)

    for cp in cps:
        cp.wait()

    ocp1 = _half_compute(N1, PPG1, EPG1, q1_out,
                         psrc1, pdst1, plen1, esrc1, edst1, smap1, q1, qv1,
                         wid, sem)
    ocp2 = _half_compute(N2, PPG2, EPG2, q2_out,
                         psrc2, pdst2, plen2, esrc2, edst2, smap2, q2, qv2,
                         wid, sem)
    ocp1.wait()
    ocp2.wait()


def _half_scratch(N, ppg, epg):
    npr = G * ppg
    ned = G * epg
    cells = G * N * N
    return [
        pltpu.VMEM((npr,), jnp.int32),
        pltpu.VMEM((npr,), jnp.int32),
        pltpu.VMEM((npr,), jnp.int32),
        pltpu.VMEM((ned,), jnp.int32),
        pltpu.VMEM((ned,), jnp.int32),
        pltpu.VMEM((cells,), jnp.int32),
        pltpu.VMEM((_qwords(epg),), jnp.float32),
        pltpu.VMEM((L, ned), jnp.float32),
    ]


_sc_counts = pl.kernel(
    _sc_body,
    out_type=(jax.ShapeDtypeStruct((L, E1), jnp.float32),
              jax.ShapeDtypeStruct((L, E2), jnp.float32)),
    mesh=plsc.VectorSubcoreMesh(core_axis_name="c", subcore_axis_name="s"),
    scratch_types=_half_scratch(N1, PPG1, EPG1) + _half_scratch(N2, PPG2, EPG2)
    + [pltpu.SemaphoreType.DMA],
    compiler_params=pltpu.CompilerParams(needs_layout_passes=False),
)


def _tc_body(q1_ref, w1_ref, q2_ref, w2_ref, o1_ref, o2_ref):
    dn = (((0,), (0,)), ((), ()))
    w1 = jnp.pad(w1_ref[...], ((0, L - 12), (0, 0)))
    w2 = jnp.pad(w2_ref[...], ((0, L - 12), (0, 0)))
    o1_ref[...] = lax.dot_general(q1_ref[...], w1, dn,
                                  preferred_element_type=jnp.float32)
    o2_ref[...] = lax.dot_general(q2_ref[...], w2, dn,
                                  preferred_element_type=jnp.float32)


_TCG = 2
_B1 = E1 // _TCG
_B2 = E2 // _TCG

_tc_project = pl.pallas_call(
    _tc_body,
    grid=(_TCG,),
    in_specs=[
        pl.BlockSpec((L, _B1), lambda i: (0, i)),
        pl.BlockSpec((12, 64), lambda i: (0, 0)),
        pl.BlockSpec((L, _B2), lambda i: (0, i)),
        pl.BlockSpec((12, 64), lambda i: (0, 0)),
    ],
    out_specs=(pl.BlockSpec((_B1, 64), lambda i: (i, 0)),
               pl.BlockSpec((_B2, 64), lambda i: (i, 0))),
    out_shape=(jax.ShapeDtypeStruct((E1, 64), jnp.float32),
               jax.ShapeDtypeStruct((E2, 64), jnp.float32)),
)


def kernel(spd_index, spd_lengths, batch, edge_index,
           e2e_spd_index, e2e_spd_lengths, e_batch, e2e_edge_index,
           W_spd, W_e2e):
    del batch, e_batch
    q1, q2 = _sc_counts(
        spd_index, spd_lengths, edge_index,
        e2e_spd_index, e2e_spd_lengths, e2e_edge_index,
    )
    return _tc_project(q1, W_spd, q2, W_e2e)


# speedup vs baseline: 1.4829x; 1.4035x over previous
import jax
import jax.numpy as jnp
from jax import lax
from jax.experimental import pallas as pl
from jax.experimental.pallas import tpu as pltpu
from jax.experimental.pallas import tpu_sc as plsc

NW = 32
NC = 2
L = 16
B = 128
G = B // NW
QSTR = 17

N1, PPG1, EPG1 = 32, 256, 64
N2, PPG2, EPG2 = 64, 512, 128
E1 = B * EPG1
E2 = B * EPG2


def _qwords(epg):
    return ((G * epg + 1) * QSTR + L - 1) // L * L


def _memset(ref, nvec, vec):
    def body(i, c):
        base = i * (4 * L)
        for j in range(4):
            ref[pl.ds(base + j * L, L)] = vec
        return c
    lax.fori_loop(0, nvec // 4, body, 0)
    for j in range(nvec // 4 * 4, nvec):
        ref[pl.ds(j * L, L)] = vec


def _half_compute(N, ppg, epg, out_h,
                  psrc, pdst, plen, esrc, edst, smap, q, qv, wid, sem):
    npr = G * ppg
    ned = G * epg
    nself = G * N
    cells = G * N * N
    ebase = wid * ned
    cell_off = wid * cells
    iota = lax.iota(jnp.int32, L)
    ones = jnp.ones((L,), jnp.float32)
    mask_n = N - 1

    def ekey(i):
        s = esrc[pl.ds(i * L, L)]
        d = edst[pl.ds(i * L, L)]
        return s * N + (d & mask_n) - cell_off

    def scat_e(i, c):
        plsc.store_scatter(smap, [ekey(i)], i * L + iota)
        return c

    lax.fori_loop(0, ned // L, scat_e, 0)

    def pair_step(i, c):
        s = psrc[pl.ds(i * L, L)]
        d = pdst[pl.ds(i * L, L)]
        t = plen[pl.ds(i * L, L)] + 1
        k = s * N + (d & mask_n) - cell_off
        slot = plsc.load_gather(smap, [k])
        plsc.addupdate_scatter(q, [slot * QSTR + t], ones)
        return c

    lax.fori_loop(0, npr // L, pair_step, 0)

    def self_step(i, c):
        iloc = i * L + iota
        k = iloc * N + (iloc & mask_n)
        slot = plsc.load_gather(smap, [k])
        plsc.addupdate_scatter(q, [slot * QSTR], ones)
        return c

    lax.fori_loop(0, nself // L, self_step, 0)

    def out_step(i, c):
        slot = plsc.load_gather(smap, [ekey(i)]) * QSTR
        for t in range(L):
            vals = plsc.load_gather(q, [slot + t])
            qv[t, pl.ds(i * L, L)] = vals
        return c

    lax.fori_loop(0, ned // L, out_step, 0)

    return pltpu.async_copy(qv, out_h.at[:, pl.ds(ebase, ned)], sem)


def _sc_body(pidx1_h, plen1_h, eidx1_h,
             pidx2_h, plen2_h, eidx2_h,
             q1_out, q2_out,
             psrc1, pdst1, plen1, esrc1, edst1, smap1, q1, qv1,
             psrc2, pdst2, plen2, esrc2, edst2, smap2, q2, qv2,
             sem):
    wid = lax.axis_index("s") * NC + lax.axis_index("c")
    npr1, ned1 = G * PPG1, G * EPG1
    npr2, ned2 = G * PPG2, G * EPG2

    cps = []
    for hbm, row, vmem, base, n in (
            (pidx1_h, 0, psrc1, wid * npr1, npr1),
            (pidx1_h, 1, pdst1, wid * npr1, npr1),
            (plen1_h, None, plen1, wid * npr1, npr1),
            (eidx1_h, 0, esrc1, wid * ned1, ned1),
            (eidx1_h, 1, edst1, wid * ned1, ned1),
            (pidx2_h, 0, psrc2, wid * npr2, npr2),
            (pidx2_h, 1, pdst2, wid * npr2, npr2),
            (plen2_h, None, plen2, wid * npr2, npr2),
            (eidx2_h, 0, esrc2, wid * ned2, ned2),
            (eidx2_h, 1, edst2, wid * ned2, ned2)):
        src = hbm.at[pl.ds(base, n)] if row is None \
            else hbm.at[row, pl.ds(base, n)]
        cps.append(pltpu.async_copy(src, vmem, sem))

    _memset(smap1, G * N1 * N1 // L, jnp.full((L,), ned1, jnp.int32))
    _memset(smap2, G * N2 * N2 // L, jnp.full((L,), ned2, jnp.int32))
    zf = jnp.zeros((L,), jnp.float32)
    _memset(q1, _qwords(EPG1) // L, zf)
    _memset(q2, _qwords(EPG2) // L, zf)

    for cp in cps:
        cp.wait()

    ocp1 = _half_compute(N1, PPG1, EPG1, q1_out,
                         psrc1, pdst1, plen1, esrc1, edst1, smap1, q1, qv1,
                         wid, sem)
    ocp2 = _half_compute(N2, PPG2, EPG2, q2_out,
                         psrc2, pdst2, plen2, esrc2, edst2, smap2, q2, qv2,
                         wid, sem)
    ocp1.wait()
    ocp2.wait()


def _half_scratch(N, ppg, epg):
    npr = G * ppg
    ned = G * epg
    cells = G * N * N
    return [
        pltpu.VMEM((npr,), jnp.int32),
        pltpu.VMEM((npr,), jnp.int32),
        pltpu.VMEM((npr,), jnp.int32),
        pltpu.VMEM((ned,), jnp.int32),
        pltpu.VMEM((ned,), jnp.int32),
        pltpu.VMEM((cells,), jnp.int32),
        pltpu.VMEM((_qwords(epg),), jnp.float32),
        pltpu.VMEM((L, ned), jnp.float32),
    ]


_sc_counts = pl.kernel(
    _sc_body,
    out_type=(jax.ShapeDtypeStruct((L, E1), jnp.float32),
              jax.ShapeDtypeStruct((L, E2), jnp.float32)),
    mesh=plsc.VectorSubcoreMesh(core_axis_name="c", subcore_axis_name="s"),
    scratch_types=_half_scratch(N1, PPG1, EPG1) + _half_scratch(N2, PPG2, EPG2)
    + [pltpu.SemaphoreType.DMA],
    compiler_params=pltpu.CompilerParams(needs_layout_passes=False),
)


def _tc_body(q1_ref, w1_ref, q2_ref, w2_ref, o1_ref, o2_ref):
    dn = (((0,), (0,)), ((), ()))
    w1 = jnp.pad(w1_ref[...], ((0, L - 12), (0, 0)))
    w2 = jnp.pad(w2_ref[...], ((0, L - 12), (0, 0)))
    o1_ref[...] = lax.dot_general(w1, q1_ref[...], dn,
                                  preferred_element_type=jnp.float32)
    o2_ref[...] = lax.dot_general(w2, q2_ref[...], dn,
                                  preferred_element_type=jnp.float32)


_TCG = 2
_B1 = E1 // _TCG
_B2 = E2 // _TCG

_tc_project = pl.pallas_call(
    _tc_body,
    grid=(_TCG,),
    in_specs=[
        pl.BlockSpec((L, _B1), lambda i: (0, i)),
        pl.BlockSpec((12, 64), lambda i: (0, 0)),
        pl.BlockSpec((L, _B2), lambda i: (0, i)),
        pl.BlockSpec((12, 64), lambda i: (0, 0)),
    ],
    out_specs=(pl.BlockSpec((64, _B1), lambda i: (0, i)),
               pl.BlockSpec((64, _B2), lambda i: (0, i))),
    out_shape=(jax.ShapeDtypeStruct((64, E1), jnp.float32),
               jax.ShapeDtypeStruct((64, E2), jnp.float32)),
)


def kernel(spd_index, spd_lengths, batch, edge_index,
           e2e_spd_index, e2e_spd_lengths, e_batch, e2e_edge_index,
           W_spd, W_e2e):
    del batch, e_batch
    q1, q2 = _sc_counts(
        spd_index, spd_lengths, edge_index,
        e2e_spd_index, e2e_spd_lengths, e2e_edge_index,
    )
    o1t, o2t = _tc_project(q1, W_spd, q2, W_e2e)
    return o1t.T, o2t.T


# speedup vs baseline: 1.5199x; 1.0249x over previous
import jax
import jax.numpy as jnp
from jax import lax
from jax.experimental import pallas as pl
from jax.experimental.pallas import tpu as pltpu
from jax.experimental.pallas import tpu_sc as plsc

NW = 32
NC = 2
L = 16
B = 128
G = B // NW
QSTR = 17
NT = 12

N1, PPG1, EPG1 = 32, 256, 64
N2, PPG2, EPG2 = 64, 512, 128
E1 = B * EPG1
E2 = B * EPG2


def _qwords(epg):
    return ((G * epg + 1) * QSTR + L - 1) // L * L


def _memset(ref, nvec, vec):
    def body(i, c):
        base = i * (4 * L)
        for j in range(4):
            ref[pl.ds(base + j * L, L)] = vec
        return c
    lax.fori_loop(0, nvec // 4, body, 0)
    for j in range(nvec // 4 * 4, nvec):
        ref[pl.ds(j * L, L)] = vec


def _half_compute(N, ppg, epg, out_h,
                  psrc, pdst, plen, esrc, edst, smap, q, qv, wid, sem):
    npr = G * ppg
    ned = G * epg
    nself = G * N
    cells = G * N * N
    ebase = wid * ned
    cell_off = wid * cells
    iota = lax.iota(jnp.int32, L)
    ones = jnp.ones((L,), jnp.float32)
    mask_n = N - 1

    def ekey(i):
        s = esrc[pl.ds(i * L, L)]
        d = edst[pl.ds(i * L, L)]
        return s * N + (d & mask_n) - cell_off

    def scat_e(i, c):
        plsc.store_scatter(smap, [ekey(i)], i * L + iota)
        return c

    lax.fori_loop(0, ned // L, scat_e, 0)

    def pair_step(i, c):
        s = psrc[pl.ds(i * L, L)]
        d = pdst[pl.ds(i * L, L)]
        t = plen[pl.ds(i * L, L)] + 1
        k = s * N + (d & mask_n) - cell_off
        slot = plsc.load_gather(smap, [k])
        plsc.addupdate_scatter(q, [slot * QSTR + t], ones)
        return c

    lax.fori_loop(0, npr // L, pair_step, 0)

    def self_step(i, c):
        iloc = i * L + iota
        k = iloc * N + (iloc & mask_n)
        slot = plsc.load_gather(smap, [k])
        plsc.addupdate_scatter(q, [slot * QSTR], ones)
        return c

    lax.fori_loop(0, nself // L, self_step, 0)

    def out_step(i, c):
        slot = plsc.load_gather(smap, [ekey(i)]) * QSTR
        for t in range(NT):
            vals = plsc.load_gather(q, [slot + t])
            qv[t, pl.ds(i * L, L)] = vals
        return c

    lax.fori_loop(0, ned // L, out_step, 0)

    return pltpu.async_copy(qv, out_h.at[:, pl.ds(ebase, ned)], sem)


def _sc_body(pidx1_h, plen1_h, eidx1_h,
             pidx2_h, plen2_h, eidx2_h,
             q1_out, q2_out,
             psrc1, pdst1, plen1, esrc1, edst1, smap1, q1, qv1,
             psrc2, pdst2, plen2, esrc2, edst2, smap2, q2, qv2,
             sem):
    wid = lax.axis_index("s") * NC + lax.axis_index("c")
    npr1, ned1 = G * PPG1, G * EPG1
    npr2, ned2 = G * PPG2, G * EPG2

    cps = []
    for hbm, row, vmem, base, n in (
            (pidx1_h, 0, psrc1, wid * npr1, npr1),
            (pidx1_h, 1, pdst1, wid * npr1, npr1),
            (plen1_h, None, plen1, wid * npr1, npr1),
            (eidx1_h, 0, esrc1, wid * ned1, ned1),
            (eidx1_h, 1, edst1, wid * ned1, ned1),
            (pidx2_h, 0, psrc2, wid * npr2, npr2),
            (pidx2_h, 1, pdst2, wid * npr2, npr2),
            (plen2_h, None, plen2, wid * npr2, npr2),
            (eidx2_h, 0, esrc2, wid * ned2, ned2),
            (eidx2_h, 1, edst2, wid * ned2, ned2)):
        src = hbm.at[pl.ds(base, n)] if row is None \
            else hbm.at[row, pl.ds(base, n)]
        cps.append(pltpu.async_copy(src, vmem, sem))

    _memset(smap1, G * N1 * N1 // L, jnp.full((L,), ned1, jnp.int32))
    _memset(smap2, G * N2 * N2 // L, jnp.full((L,), ned2, jnp.int32))
    zf = jnp.zeros((L,), jnp.float32)
    _memset(q1, _qwords(EPG1) // L, zf)
    _memset(q2, _qwords(EPG2) // L, zf)

    for cp in cps:
        cp.wait()

    ocp1 = _half_compute(N1, PPG1, EPG1, q1_out,
                         psrc1, pdst1, plen1, esrc1, edst1, smap1, q1, qv1,
                         wid, sem)
    ocp2 = _half_compute(N2, PPG2, EPG2, q2_out,
                         psrc2, pdst2, plen2, esrc2, edst2, smap2, q2, qv2,
                         wid, sem)
    ocp1.wait()
    ocp2.wait()


def _half_scratch(N, ppg, epg):
    npr = G * ppg
    ned = G * epg
    cells = G * N * N
    return [
        pltpu.VMEM((npr,), jnp.int32),
        pltpu.VMEM((npr,), jnp.int32),
        pltpu.VMEM((npr,), jnp.int32),
        pltpu.VMEM((ned,), jnp.int32),
        pltpu.VMEM((ned,), jnp.int32),
        pltpu.VMEM((cells,), jnp.int32),
        pltpu.VMEM((_qwords(epg),), jnp.float32),
        pltpu.VMEM((NT, ned), jnp.float32),
    ]


_sc_counts = pl.kernel(
    _sc_body,
    out_type=(jax.ShapeDtypeStruct((NT, E1), jnp.float32),
              jax.ShapeDtypeStruct((NT, E2), jnp.float32)),
    mesh=plsc.VectorSubcoreMesh(core_axis_name="c", subcore_axis_name="s"),
    scratch_types=_half_scratch(N1, PPG1, EPG1) + _half_scratch(N2, PPG2, EPG2)
    + [pltpu.SemaphoreType.DMA],
    compiler_params=pltpu.CompilerParams(needs_layout_passes=False),
)


def _tc_body(q1_ref, w1_ref, q2_ref, w2_ref, o1_ref, o2_ref):
    dn = (((0,), (0,)), ((), ()))
    o1_ref[...] = lax.dot_general(w1_ref[...], q1_ref[...], dn,
                                  preferred_element_type=jnp.float32)
    o2_ref[...] = lax.dot_general(w2_ref[...], q2_ref[...], dn,
                                  preferred_element_type=jnp.float32)


_TCG = 2
_B1 = E1 // _TCG
_B2 = E2 // _TCG

_tc_project = pl.pallas_call(
    _tc_body,
    grid=(_TCG,),
    in_specs=[
        pl.BlockSpec((NT, _B1), lambda i: (0, i)),
        pl.BlockSpec((12, 64), lambda i: (0, 0)),
        pl.BlockSpec((NT, _B2), lambda i: (0, i)),
        pl.BlockSpec((12, 64), lambda i: (0, 0)),
    ],
    out_specs=(pl.BlockSpec((64, _B1), lambda i: (0, i)),
               pl.BlockSpec((64, _B2), lambda i: (0, i))),
    out_shape=(jax.ShapeDtypeStruct((64, E1), jnp.float32),
               jax.ShapeDtypeStruct((64, E2), jnp.float32)),
)


def kernel(spd_index, spd_lengths, batch, edge_index,
           e2e_spd_index, e2e_spd_lengths, e_batch, e2e_edge_index,
           W_spd, W_e2e):
    del batch, e_batch
    q1, q2 = _sc_counts(
        spd_index, spd_lengths, edge_index,
        e2e_spd_index, e2e_spd_lengths, e2e_edge_index,
    )
    o1t, o2t = _tc_project(q1, W_spd, q2, W_e2e)
    return o1t.T, o2t.T
